# baseline (speedup 1.0000x reference)
import functools

import jax
import jax.numpy as jnp
from jax.experimental import pallas as pl

B, N, D, H, E, K, U = 4, 1024, 256, 256, 5, 2, 128
EP = 128

_NEG_INF = float('-inf')


def _moge_kernel(adj_ref, x_ref, unk_ref,
                 fc1_w_ref, fc1_b_ref, fc2_w_ref, fc2_b_ref,
                 wm_ref, bm_ref, ww_ref, bw_ref, wxn_ref, bx_ref,
                 bn_ref, wd_ref, bd_ref,
                 out_ref):
    adj = adj_ref[0]
    x = x_ref[0]
    unk = unk_ref[0]

    f32 = jnp.float32
    bf16 = jnp.bfloat16
    dot = functools.partial(jnp.dot, preferred_element_type=f32)

    node_ids = jax.lax.broadcasted_iota(jnp.int32, (N, U), 0)
    hit = jnp.any(node_ids == unk, axis=1, keepdims=True)
    known = jnp.where(hit, f32(0.0), f32(1.0))
    x_m = x * known
    x_mb = x_m.astype(bf16)

    deg = jnp.sum(adj, axis=1, keepdims=True) + f32(1e-6)
    inv_deg = f32(1.0) / deg
    adjb = adj.astype(bf16)

    S = dot(adjb, x_mb) * inv_deg
    Sb = S.astype(bf16)
    h2 = dot(adjb, Sb) * inv_deg
    e_adj = jnp.exp(adj)
    esum = jnp.sum(e_adj, axis=1, keepdims=True)
    Wt = dot(e_adj.astype(bf16), x_mb) / esum

    hg = jnp.maximum(dot(x, fc1_w_ref[...]) + fc1_b_ref[...], f32(0.0))
    logits = dot(hg, fc2_w_ref[...]) + fc2_b_ref[...]
    col = jax.lax.broadcasted_iota(jnp.int32, (N, EP), 1)
    valid = col < E
    l = jnp.where(valid, logits, _NEG_INF)
    m1 = jnp.max(l, axis=1, keepdims=True)
    idx1 = jnp.min(jnp.where(l == m1, col, EP), axis=1, keepdims=True)
    l2 = jnp.where(col == idx1, _NEG_INF, l)
    m2 = jnp.max(l2, axis=1, keepdims=True)
    idx2 = jnp.min(jnp.where(l2 == m2, col, EP), axis=1, keepdims=True)
    topk_mask = (col == idx1) | (col == idx2)
    sl = jnp.where(valid, jnp.where(topk_mask, l, f32(0.0)), _NEG_INF)
    smax = jnp.max(sl, axis=1, keepdims=True)
    eg = jnp.exp(sl - smax)
    g = eg / jnp.sum(eg, axis=1, keepdims=True)

    relu = lambda v: jnp.maximum(v, f32(0.0))
    mean_out = relu(dot(Sb, wm_ref[...]) + bm_ref[...])
    wmean_out = relu(dot(Wt.astype(bf16), ww_ref[...]) + bw_ref[...])
    pxn = dot(x_mb, wxn_ref[...])
    px = relu(pxn[:, :H] + bx_ref[...])
    max_out = jnp.max(px, axis=0, keepdims=True)
    pn = relu(pxn[:, H:] + bn_ref[...])
    min_out = jnp.min(pn, axis=0, keepdims=True)
    diff = f32(0.9) * x_m + f32(0.05) * (S + h2)
    diff_out = relu(dot(diff.astype(bf16), wd_ref[...]) + bd_ref[...])

    out = (g[:, 0:1] * mean_out
           + g[:, 1:2] * wmean_out
           + g[:, 2:3] * max_out
           + g[:, 3:4] * min_out
           + g[:, 4:5] * diff_out)
    out_ref[0] = out


def kernel(x_enc, adj, batch_unknown_nodes, fc1_w, fc1_b, fc2_w, fc2_b,
           Wm, bm, Ww, bw, Wx, bx, Wn, bn, Wd, bd):
    f32 = jnp.float32
    bf16 = jnp.bfloat16
    unk = batch_unknown_nodes.astype(jnp.int32).reshape(B, 1, U)
    fc2_wp = jnp.zeros((D, EP), f32).at[:, :E].set(fc2_w)
    fc2_bp = jnp.zeros((1, EP), f32).at[0, :E].set(fc2_b)
    Wxn = jnp.concatenate([Wx, Wn], axis=1).astype(bf16)

    def row(v):
        return v.reshape(1, -1).astype(f32)

    full = lambda shape: pl.BlockSpec(shape, lambda b: (0,) * len(shape))
    batched = lambda shape: pl.BlockSpec(shape, lambda b: (b,) + (0,) * (len(shape) - 1))

    out = pl.pallas_call(
        _moge_kernel,
        grid=(B,),
        in_specs=[
            batched((1, N, N)),
            batched((1, N, D)),
            batched((1, 1, U)),
            full((D, H)), full((1, H)),
            full((D, EP)), full((1, EP)),
            full((D, H)), full((1, H)),
            full((D, H)), full((1, H)),
            full((D, 2 * H)), full((1, H)),
            full((1, H)),
            full((D, H)), full((1, H)),
        ],
        out_specs=batched((1, N, H)),
        out_shape=jax.ShapeDtypeStruct((B, N, H), f32),
    )(adj, x_enc, unk,
      fc1_w, row(fc1_b), fc2_wp, fc2_bp,
      Wm.astype(bf16), row(bm), Ww.astype(bf16), row(bw),
      Wxn, row(bx), row(bn), Wd.astype(bf16), row(bd))
    return out


# speedup vs baseline: 1.1491x; 1.1491x over previous
import functools

import jax
import jax.numpy as jnp
from jax.experimental import pallas as pl

B, N, D, H, E, K, U = 4, 1024, 256, 256, 5, 2, 128
EP = 128

_NEG_INF = float('-inf')


def _moge_kernel(adj_ref, x_ref, unk_ref,
                 fc1_w_ref, fc1_b_ref, fc2_w_ref, fc2_b_ref,
                 wm_ref, bm_ref, ww_ref, bw_ref, wxn_ref, bx_ref,
                 bn_ref, wd_ref, bd_ref,
                 out_ref):
    adj = adj_ref[0]
    x = x_ref[0]
    unk = unk_ref[0]

    f32 = jnp.float32
    dot = functools.partial(jnp.dot, preferred_element_type=f32)

    node_ids = jax.lax.broadcasted_iota(jnp.int32, (N, U), 0)
    hit = jnp.any(node_ids == unk, axis=1, keepdims=True)
    known = jnp.where(hit, f32(0.0), f32(1.0))
    x_m = x * known

    deg = jnp.sum(adj, axis=1, keepdims=True) + f32(1e-6)
    inv_deg = f32(1.0) / deg

    S = dot(adj, x_m) * inv_deg
    h2 = dot(adj, S) * inv_deg
    e_adj = jnp.exp(adj)
    esum = jnp.sum(e_adj, axis=1, keepdims=True)
    Wt = dot(e_adj, x_m) / esum

    hg = jnp.maximum(dot(x, fc1_w_ref[...]) + fc1_b_ref[...], f32(0.0))
    logits = dot(hg, fc2_w_ref[...]) + fc2_b_ref[...]
    col = jax.lax.broadcasted_iota(jnp.int32, (N, EP), 1)
    valid = col < E
    l = jnp.where(valid, logits, _NEG_INF)
    m1 = jnp.max(l, axis=1, keepdims=True)
    idx1 = jnp.min(jnp.where(l == m1, col, EP), axis=1, keepdims=True)
    l2 = jnp.where(col == idx1, _NEG_INF, l)
    m2 = jnp.max(l2, axis=1, keepdims=True)
    idx2 = jnp.min(jnp.where(l2 == m2, col, EP), axis=1, keepdims=True)
    topk_mask = (col == idx1) | (col == idx2)
    sl = jnp.where(valid, jnp.where(topk_mask, l, f32(0.0)), _NEG_INF)
    smax = jnp.max(sl, axis=1, keepdims=True)
    eg = jnp.exp(sl - smax)
    g = eg / jnp.sum(eg, axis=1, keepdims=True)

    relu = lambda v: jnp.maximum(v, f32(0.0))
    mean_out = relu(dot(S, wm_ref[...]) + bm_ref[...])
    wmean_out = relu(dot(Wt, ww_ref[...]) + bw_ref[...])
    pxn = dot(x_m, wxn_ref[...])
    px = relu(pxn[:, :H] + bx_ref[...])
    max_out = jnp.max(px, axis=0, keepdims=True)
    pn = relu(pxn[:, H:] + bn_ref[...])
    min_out = jnp.min(pn, axis=0, keepdims=True)
    diff = f32(0.9) * x_m + f32(0.05) * (S + h2)
    diff_out = relu(dot(diff, wd_ref[...]) + bd_ref[...])

    out = (g[:, 0:1] * mean_out
           + g[:, 1:2] * wmean_out
           + g[:, 2:3] * max_out
           + g[:, 3:4] * min_out
           + g[:, 4:5] * diff_out)
    out_ref[0] = out


def kernel(x_enc, adj, batch_unknown_nodes, fc1_w, fc1_b, fc2_w, fc2_b,
           Wm, bm, Ww, bw, Wx, bx, Wn, bn, Wd, bd):
    f32 = jnp.float32
    unk = batch_unknown_nodes.astype(jnp.int32).reshape(B, 1, U)
    fc2_wp = jnp.zeros((D, EP), f32).at[:, :E].set(fc2_w)
    fc2_bp = jnp.zeros((1, EP), f32).at[0, :E].set(fc2_b)
    Wxn = jnp.concatenate([Wx, Wn], axis=1)

    def row(v):
        return v.reshape(1, -1).astype(f32)

    full = lambda shape: pl.BlockSpec(shape, lambda b: (0,) * len(shape))
    batched = lambda shape: pl.BlockSpec(shape, lambda b: (b,) + (0,) * (len(shape) - 1))

    out = pl.pallas_call(
        _moge_kernel,
        grid=(B,),
        in_specs=[
            batched((1, N, N)),
            batched((1, N, D)),
            batched((1, 1, U)),
            full((D, H)), full((1, H)),
            full((D, EP)), full((1, EP)),
            full((D, H)), full((1, H)),
            full((D, H)), full((1, H)),
            full((D, 2 * H)), full((1, H)),
            full((1, H)),
            full((D, H)), full((1, H)),
        ],
        out_specs=batched((1, N, H)),
        out_shape=jax.ShapeDtypeStruct((B, N, H), f32),
    )(adj, x_enc, unk,
      fc1_w, row(fc1_b), fc2_wp, fc2_bp,
      Wm, row(bm), Ww, row(bw),
      Wxn, row(bx), row(bn), Wd, row(bd))
    return out


# speedup vs baseline: 1.5991x; 1.3916x over previous
import functools

import jax
import jax.numpy as jnp
from jax.experimental import pallas as pl

B, N, D, H, E, K, U = 4, 1024, 256, 256, 5, 2, 128
EP = 128

_NEG_INF = float('-inf')


def _moge_kernel(adj_ref, x_ref, unk_ref,
                 fc1_w_ref, fc1_b_ref, fc2_w_ref, fc2_b_ref,
                 wm_ref, bm_ref, ww_ref, bw_ref, wx_ref, bx_ref,
                 wn_ref, bn_ref, wd_ref, bd_ref,
                 out_ref):
    adj = adj_ref[0]
    x = x_ref[0]
    unk = unk_ref[0]

    f32 = jnp.float32
    dot = functools.partial(jnp.dot, preferred_element_type=f32)

    node_ids = jax.lax.broadcasted_iota(jnp.int32, (N, U), 0)
    hit = jnp.any(node_ids == unk, axis=1, keepdims=True)
    known = jnp.where(hit, f32(0.0), f32(1.0))
    x_m = x * known

    deg = jnp.sum(adj, axis=1, keepdims=True) + f32(1e-6)
    inv_deg = f32(1.0) / deg

    S = dot(adj, x_m) * inv_deg
    h2 = dot(adj, S) * inv_deg
    e_adj = jnp.exp(adj)
    esum = jnp.sum(e_adj, axis=1, keepdims=True)
    Wt = dot(e_adj, x_m) / esum

    hg = jnp.maximum(dot(x, fc1_w_ref[...]) + fc1_b_ref[...], f32(0.0))
    l = dot(hg, fc2_w_ref[...]) + fc2_b_ref[...]
    col = jax.lax.broadcasted_iota(jnp.int32, (N, E), 1)
    m1 = jnp.max(l, axis=1, keepdims=True)
    idx1 = jnp.min(jnp.where(l == m1, col, E), axis=1, keepdims=True)
    l2 = jnp.where(col == idx1, _NEG_INF, l)
    m2 = jnp.max(l2, axis=1, keepdims=True)
    idx2 = jnp.min(jnp.where(l2 == m2, col, E), axis=1, keepdims=True)
    e1 = jnp.exp(m1)
    e2 = jnp.exp(m2)
    inv_z = f32(1.0) / (e1 + e2 + f32(E - K))

    def gate(e):
        return inv_z * jnp.where(idx1 == e, e1, jnp.where(idx2 == e, e2, f32(1.0)))

    relu = lambda v: jnp.maximum(v, f32(0.0))
    mean_out = relu(dot(S, wm_ref[...]) + bm_ref[...])
    wmean_out = relu(dot(Wt, ww_ref[...]) + bw_ref[...])
    px = relu(dot(x_m, wx_ref[...]) + bx_ref[...])
    max_out = jnp.max(px, axis=0, keepdims=True)
    pn = relu(dot(x_m, wn_ref[...]) + bn_ref[...])
    min_out = jnp.min(pn, axis=0, keepdims=True)
    diff = f32(0.9) * x_m + f32(0.05) * (S + h2)
    diff_out = relu(dot(diff, wd_ref[...]) + bd_ref[...])

    out = (gate(0) * mean_out
           + gate(1) * wmean_out
           + gate(2) * max_out
           + gate(3) * min_out
           + gate(4) * diff_out)
    out_ref[0] = out


def kernel(x_enc, adj, batch_unknown_nodes, fc1_w, fc1_b, fc2_w, fc2_b,
           Wm, bm, Ww, bw, Wx, bx, Wn, bn, Wd, bd):
    f32 = jnp.float32
    unk = batch_unknown_nodes.astype(jnp.int32).reshape(B, 1, U)

    def row(v):
        return v.reshape(1, -1).astype(f32)

    full = lambda shape: pl.BlockSpec(shape, lambda b: (0,) * len(shape))
    batched = lambda shape: pl.BlockSpec(shape, lambda b: (b,) + (0,) * (len(shape) - 1))

    out = pl.pallas_call(
        _moge_kernel,
        grid=(B,),
        in_specs=[
            batched((1, N, N)),
            batched((1, N, D)),
            batched((1, 1, U)),
            full((D, H)), full((1, H)),
            full((D, E)), full((1, E)),
            full((D, H)), full((1, H)),
            full((D, H)), full((1, H)),
            full((D, H)), full((1, H)),
            full((D, H)), full((1, H)),
            full((D, H)), full((1, H)),
        ],
        out_specs=batched((1, N, H)),
        out_shape=jax.ShapeDtypeStruct((B, N, H), f32),
    )(adj, x_enc, unk,
      fc1_w, row(fc1_b), fc2_w, row(fc2_b),
      Wm, row(bm), Ww, row(bw), Wx, row(bx), Wn, row(bn), Wd, row(bd))
    return out


# speedup vs baseline: 1.6843x; 1.0533x over previous
import functools

import jax
import jax.numpy as jnp
from jax.experimental import pallas as pl

B, N, D, H, E, K, U = 4, 1024, 256, 256, 5, 2, 128
EP = 128

_NEG_INF = float('-inf')


def _moge_kernel(adj_ref, x_ref, unk_ref,
                 fc1_w_ref, fc1_b_ref, fc2_w_ref, fc2_b_ref,
                 wm_ref, bm_ref, ww_ref, bw_ref, wx_ref, bx_ref,
                 wn_ref, bn_ref, wd_ref, bd_ref,
                 out_ref):
    adj = adj_ref[0]
    x = x_ref[0]
    unk = unk_ref[0]

    f32 = jnp.float32
    dot = functools.partial(jnp.dot, preferred_element_type=f32)

    node_ids = jax.lax.broadcasted_iota(jnp.int32, (N, U), 0)
    hit = jnp.any(node_ids == unk, axis=1, keepdims=True)
    known = jnp.where(hit, f32(0.0), f32(1.0))
    x_m = x * known

    deg = jnp.sum(adj, axis=1, keepdims=True) + f32(1e-6)
    inv_deg = f32(1.0) / deg

    S = dot(adj, x_m) * inv_deg
    h2 = dot(adj, S) * inv_deg
    e_adj = jnp.exp(adj)
    esum = jnp.sum(e_adj, axis=1, keepdims=True)
    Wt = dot(e_adj, x_m) / esum

    hg = jnp.maximum(dot(x, fc1_w_ref[...]) + fc1_b_ref[...], f32(0.0))
    l = dot(hg, fc2_w_ref[...]) + fc2_b_ref[...]
    lT = l.T
    rowT = jax.lax.broadcasted_iota(jnp.int32, (E, N), 0)
    m1 = jnp.max(lT, axis=0, keepdims=True)
    idx1 = jnp.min(jnp.where(lT == m1, rowT, E), axis=0, keepdims=True)
    l2 = jnp.where(rowT == idx1, _NEG_INF, lT)
    m2 = jnp.max(l2, axis=0, keepdims=True)
    idx2 = jnp.min(jnp.where(l2 == m2, rowT, E), axis=0, keepdims=True)
    e1 = jnp.exp(m1)
    e2 = jnp.exp(m2)
    inv_z = f32(1.0) / (e1 + e2 + f32(E - K))
    gT = inv_z * jnp.where(rowT == idx1, e1,
                           jnp.where(rowT == idx2, e2, f32(1.0)))
    g = gT.T

    def gate(e):
        return g[:, e:e + 1]

    relu = lambda v: jnp.maximum(v, f32(0.0))
    mean_out = relu(dot(S, wm_ref[...]) + bm_ref[...])
    wmean_out = relu(dot(Wt, ww_ref[...]) + bw_ref[...])
    px = relu(dot(x_m, wx_ref[...]) + bx_ref[...])
    max_out = jnp.max(px, axis=0, keepdims=True)
    pn = relu(dot(x_m, wn_ref[...]) + bn_ref[...])
    min_out = jnp.min(pn, axis=0, keepdims=True)
    diff = f32(0.9) * x_m + f32(0.05) * (S + h2)
    diff_out = relu(dot(diff, wd_ref[...]) + bd_ref[...])

    out = (gate(0) * mean_out
           + gate(1) * wmean_out
           + gate(2) * max_out
           + gate(3) * min_out
           + gate(4) * diff_out)
    out_ref[0] = out


def kernel(x_enc, adj, batch_unknown_nodes, fc1_w, fc1_b, fc2_w, fc2_b,
           Wm, bm, Ww, bw, Wx, bx, Wn, bn, Wd, bd):
    f32 = jnp.float32
    unk = batch_unknown_nodes.astype(jnp.int32).reshape(B, 1, U)

    def row(v):
        return v.reshape(1, -1).astype(f32)

    full = lambda shape: pl.BlockSpec(shape, lambda b: (0,) * len(shape))
    batched = lambda shape: pl.BlockSpec(shape, lambda b: (b,) + (0,) * (len(shape) - 1))

    out = pl.pallas_call(
        _moge_kernel,
        grid=(B,),
        in_specs=[
            batched((1, N, N)),
            batched((1, N, D)),
            batched((1, 1, U)),
            full((D, H)), full((1, H)),
            full((D, E)), full((1, E)),
            full((D, H)), full((1, H)),
            full((D, H)), full((1, H)),
            full((D, H)), full((1, H)),
            full((D, H)), full((1, H)),
            full((D, H)), full((1, H)),
        ],
        out_specs=batched((1, N, H)),
        out_shape=jax.ShapeDtypeStruct((B, N, H), f32),
    )(adj, x_enc, unk,
      fc1_w, row(fc1_b), fc2_w, row(fc2_b),
      Wm, row(bm), Ww, row(bw), Wx, row(bx), Wn, row(bn), Wd, row(bd))
    return out


# speedup vs baseline: 1.7018x; 1.0104x over previous
import functools

import jax
import jax.numpy as jnp
from jax.experimental import pallas as pl

B, N, D, H, E, K, U = 4, 1024, 256, 256, 5, 2, 128
EP = 128

_NEG_INF = float('-inf')


def _moge_kernel(adj_ref, x_ref, unk_ref,
                 fc1_w_ref, fc1_b_ref, fc2_w_ref, fc2_b_ref,
                 wm_ref, bm_ref, ww_ref, bw_ref, wx_ref, bx_ref,
                 wn_ref, bn_ref, wd_ref, bd_ref,
                 out_ref):
    adj = adj_ref[0]
    x = x_ref[0]
    unk = unk_ref[0]

    f32 = jnp.float32
    dot = functools.partial(jnp.dot, preferred_element_type=f32)

    node_ids = jax.lax.broadcasted_iota(jnp.int32, (N, U), 0)
    hit = jnp.any(node_ids == unk, axis=1, keepdims=True)
    known = jnp.where(hit, f32(0.0), f32(1.0))
    x_m = x * known

    deg = jnp.sum(adj, axis=1, keepdims=True) + f32(1e-6)
    inv_deg = f32(1.0) / deg

    S = dot(adj, x_m) * inv_deg
    h2 = dot(adj, S) * inv_deg
    e_adj = jnp.exp(adj)
    esum = jnp.sum(e_adj, axis=1, keepdims=True)
    Wt = dot(e_adj, x_m) / esum

    hg = jnp.maximum(dot(x, fc1_w_ref[...]) + fc1_b_ref[...], f32(0.0))
    l = dot(hg, fc2_w_ref[...]) + fc2_b_ref[...]
    lT = l.T
    rowT = jax.lax.broadcasted_iota(jnp.int32, (E, N), 0)
    m1 = jnp.max(lT, axis=0, keepdims=True)
    idx1 = jnp.min(jnp.where(lT == m1, rowT, E), axis=0, keepdims=True)
    l2 = jnp.where(rowT == idx1, _NEG_INF, lT)
    m2 = jnp.max(l2, axis=0, keepdims=True)
    idx2 = jnp.min(jnp.where(l2 == m2, rowT, E), axis=0, keepdims=True)
    e1 = jnp.exp(m1)
    e2 = jnp.exp(m2)
    inv_z = f32(1.0) / (e1 + e2 + f32(E - K))
    gT = inv_z * jnp.where(rowT == idx1, e1,
                           jnp.where(rowT == idx2, e2, f32(1.0)))
    g = gT.T

    def gate(e):
        return g[:, e:e + 1]

    relu = lambda v: jnp.maximum(v, f32(0.0))
    mean_out = relu(dot(S, wm_ref[...]) + bm_ref[...])
    wmean_out = relu(dot(Wt, ww_ref[...]) + bw_ref[...])
    px = relu(dot(x_m, wx_ref[...]) + bx_ref[...])
    max_out = jnp.max(px, axis=0, keepdims=True)
    pn = relu(dot(x_m, wn_ref[...]) + bn_ref[...])
    min_out = jnp.min(pn, axis=0, keepdims=True)
    diff = f32(0.9) * x_m + f32(0.05) * (S + h2)
    diff_out = relu(dot(diff, wd_ref[...]) + bd_ref[...])

    pool_rows = jnp.concatenate([max_out, min_out], axis=0)
    out = (gate(0) * mean_out
           + gate(1) * wmean_out
           + gate(4) * diff_out
           + dot(g[:, 2:4], pool_rows))
    out_ref[0] = out


def kernel(x_enc, adj, batch_unknown_nodes, fc1_w, fc1_b, fc2_w, fc2_b,
           Wm, bm, Ww, bw, Wx, bx, Wn, bn, Wd, bd):
    f32 = jnp.float32
    unk = batch_unknown_nodes.astype(jnp.int32).reshape(B, 1, U)

    def row(v):
        return v.reshape(1, -1).astype(f32)

    full = lambda shape: pl.BlockSpec(shape, lambda b: (0,) * len(shape))
    batched = lambda shape: pl.BlockSpec(shape, lambda b: (b,) + (0,) * (len(shape) - 1))

    out = pl.pallas_call(
        _moge_kernel,
        grid=(B,),
        in_specs=[
            batched((1, N, N)),
            batched((1, N, D)),
            batched((1, 1, U)),
            full((D, H)), full((1, H)),
            full((D, E)), full((1, E)),
            full((D, H)), full((1, H)),
            full((D, H)), full((1, H)),
            full((D, H)), full((1, H)),
            full((D, H)), full((1, H)),
            full((D, H)), full((1, H)),
        ],
        out_specs=batched((1, N, H)),
        out_shape=jax.ShapeDtypeStruct((B, N, H), f32),
    )(adj, x_enc, unk,
      fc1_w, row(fc1_b), fc2_w, row(fc2_b),
      Wm, row(bm), Ww, row(bw), Wx, row(bx), Wn, row(bn), Wd, row(bd))
    return out
